# baseline (speedup 1.0000x reference)
import jax
import jax.numpy as jnp
from jax.experimental import pallas as pl

K, T, H, B, F = 64, 32, 64, 256, 1
KB = K * B


def _ka_body(h_ref, c_ref, enc_ref, W1T_ref, b1_ref, w2T_ref, b2_ref,
             o_hbar, o_cbar, o_a):
    h = h_ref[...]
    c = c_ref[...]
    enc3 = enc_ref[...]
    hbar = jnp.mean(h.reshape(K, B, H), axis=0)
    cbar = jnp.mean(c.reshape(K, B, H), axis=0)
    o_hbar[...] = hbar
    o_cbar[...] = cbar
    x2 = jnp.concatenate([
        jnp.broadcast_to(hbar[:, None, :], (B, T, H)),
        jnp.broadcast_to(cbar[:, None, :], (B, T, H)),
        enc3], axis=2).reshape(B * T, 3 * H)
    mm = jnp.dot(x2, W1T_ref[...], preferred_element_type=jnp.float32)
    a1 = jnp.tanh(mm + b1_ref[...])
    o_a[...] = jnp.dot(a1, w2T_ref[...], preferred_element_type=jnp.float32) + b2_ref[0, 0]


def _kb_body(h_ref, c_ref, context_ref, ytp_ref, eps_ref, hbar_ref,
             fcwT_ref, fcb_ref, varWT_ref, varb_ref,
             wiT_ref, bi_ref, WhT_ref, bh_ref, wdecT_ref, bdec_ref, pdfwT_ref, pdfb_ref,
             o_h, o_c, o_proj, o_lpun):
    RB = 2048
    KB_ = RB // B
    h = h_ref[...]
    c = c_ref[...]
    cc = jnp.concatenate([context_ref[...], ytp_ref[...]], axis=1)
    ytilde = jnp.dot(cc, fcwT_ref[...], preferred_element_type=jnp.float32) + fcb_ref[0, 0]
    vv = jnp.concatenate([ytilde, hbar_ref[...]], axis=1)
    var = jnp.dot(vv, varWT_ref[...], preferred_element_type=jnp.float32) + varb_ref[...]
    ytb = jnp.broadcast_to(ytilde.reshape(1, B, 1), (KB_, B, 1)).reshape(RB, 1)
    gates = ((ytb * wiT_ref[...] + bi_ref[...])
             + jnp.dot(h, WhT_ref[...], preferred_element_type=jnp.float32)) + bh_ref[...]
    i_g = jax.nn.sigmoid(gates[:, 0:H])
    f_g = jax.nn.sigmoid(gates[:, H:2 * H])
    g_g = jnp.tanh(gates[:, 2 * H:3 * H])
    o_g = jax.nn.sigmoid(gates[:, 3 * H:4 * H])
    cells = f_g * c + i_g * g_g
    h_new = o_g * jnp.tanh(cells)
    o_c[...] = cells
    std = jax.nn.softplus(jnp.broadcast_to(var[None, :, :], (KB_, B, H)).reshape(RB, H))
    hid = h_new + eps_ref[...] * std
    o_h[...] = hid
    o_proj[...] = jnp.dot(hid, wdecT_ref[...], preferred_element_type=jnp.float32) + bdec_ref[0, 0]
    yy2 = jnp.concatenate([hid, ytb], axis=1)
    o_lpun[...] = jnp.dot(yy2, pdfwT_ref[...], preferred_element_type=jnp.float32) + pdfb_ref[0, 0]


def _kc_body(projkb_ref, lpunkb_ref, gum_ref, o_comp, o_idx2):
    KC = 16
    pc = projkb_ref[...]
    kiota = jax.lax.broadcasted_iota(jnp.int32, (K, K, B), 0)
    jiota = jax.lax.broadcasted_iota(jnp.int32, (K, K, B), 1)
    lt = pc[None, :, :] < pc[:, None, :]
    eq = (pc[None, :, :] == pc[:, None, :]) & (jiota < kiota)
    rank = jnp.sum((lt | eq).astype(jnp.int32), axis=1)
    onehot = rank[None, :, :] == kiota
    perm = jnp.sum(jnp.where(onehot, jiota, 0), axis=1)
    lpsort = jnp.sum(jnp.where(onehot, lpunkb_ref[...][None, :, :], 0.0), axis=1)
    prob_bk = jnp.exp(lpsort).T
    p3 = prob_bk.reshape(4, 64, K)
    denom = jnp.sum(p3, axis=1, keepdims=True)
    w_bk = (p3 / denom).reshape(B, K)
    logits = jnp.log(w_bk + 1e-12)
    scores = logits[None, :, :] + gum_ref[...]
    m = jnp.max(scores, axis=2, keepdims=True)
    jiota3 = jax.lax.broadcasted_iota(jnp.int32, (KC, B, K), 2)
    idx2 = jnp.min(jnp.where(scores == m, jiota3, K), axis=2)
    o_idx2[...] = idx2
    sel = idx2[:, None, :] == jax.lax.broadcasted_iota(jnp.int32, (KC, K, B), 1)
    o_comp[...] = jnp.sum(jnp.where(sel, perm[None, :, :], 0), axis=1)


def _f32(shape):
    return jax.ShapeDtypeStruct(shape, jnp.float32)


def _i32(shape):
    return jax.ShapeDtypeStruct(shape, jnp.int32)


_ka = pl.pallas_call(
    _ka_body,
    out_shape=(_f32((B, H)), _f32((B, H)), _f32((B * T, 1))),
)

_RB = 2048
_NB = KB // _RB

_kb = pl.pallas_call(
    _kb_body,
    grid=(_NB,),
    in_specs=[
        pl.BlockSpec((_RB, H), lambda i: (i, 0)),
        pl.BlockSpec((_RB, H), lambda i: (i, 0)),
        pl.BlockSpec((B, H), lambda i: (0, 0)),
        pl.BlockSpec((B, 1), lambda i: (0, 0)),
        pl.BlockSpec((_RB, H), lambda i: (i, 0)),
        pl.BlockSpec((B, H), lambda i: (0, 0)),
        pl.BlockSpec((H + 1, 1), lambda i: (0, 0)),
        pl.BlockSpec((1, 1), lambda i: (0, 0)),
        pl.BlockSpec((H + 1, H), lambda i: (0, 0)),
        pl.BlockSpec((1, H), lambda i: (0, 0)),
        pl.BlockSpec((1, 4 * H), lambda i: (0, 0)),
        pl.BlockSpec((1, 4 * H), lambda i: (0, 0)),
        pl.BlockSpec((H, 4 * H), lambda i: (0, 0)),
        pl.BlockSpec((1, 4 * H), lambda i: (0, 0)),
        pl.BlockSpec((H, 1), lambda i: (0, 0)),
        pl.BlockSpec((1, 1), lambda i: (0, 0)),
        pl.BlockSpec((H + 1, 1), lambda i: (0, 0)),
        pl.BlockSpec((1, 1), lambda i: (0, 0)),
    ],
    out_specs=(
        pl.BlockSpec((_RB, H), lambda i: (i, 0)),
        pl.BlockSpec((_RB, H), lambda i: (i, 0)),
        pl.BlockSpec((_RB, 1), lambda i: (i, 0)),
        pl.BlockSpec((_RB, 1), lambda i: (i, 0)),
    ),
    out_shape=(_f32((KB, H)), _f32((KB, H)), _f32((KB, 1)), _f32((KB, 1))),
)

_KC = 16

_kc = pl.pallas_call(
    _kc_body,
    grid=(K // _KC,),
    in_specs=[
        pl.BlockSpec((K, B), lambda i: (0, 0)),
        pl.BlockSpec((K, B), lambda i: (0, 0)),
        pl.BlockSpec((_KC, B, K), lambda i: (i, 0, 0)),
    ],
    out_specs=(
        pl.BlockSpec((_KC, B), lambda i: (i, 0)),
        pl.BlockSpec((_KC, B), lambda i: (i, 0)),
    ),
    out_shape=(_i32((K, B)), _i32((K, B))),
)


def kernel(input_encoded, y_prev, attn_W1, attn_b1, attn_W2, attn_b2, lstm_Wi, lstm_Wh, lstm_bi, lstm_bh, fc_W, fc_b, fcdec_W, fcdec_b, fcenc_W, fcenc_b, var_W, var_b, pdf_W, pdf_b):
    base = jax.random.key(42)
    Bn = input_encoded.shape[0]
    eps_all = [jax.random.normal(jax.random.fold_in(base, 2 * t), (K * Bn, H), dtype=jnp.float32) for t in range(T)]
    gum_all = [jax.random.gumbel(jax.random.fold_in(base, 2 * t + 1), (K, Bn, K), dtype=jnp.float32) for t in range(T)]

    W1T = attn_W1.T
    b1r = attn_b1.reshape(1, H)
    w2T = attn_W2.T
    b2r = attn_b2.reshape(1, 1)
    fcwT = fc_W.T
    fcbr = fc_b.reshape(1, 1)
    varWT = var_W.T
    varbr = var_b.reshape(1, H)
    wiTr = lstm_Wi.T
    bir = lstm_bi.reshape(1, 4 * H)
    WhT = lstm_Wh.T
    bhr = lstm_bh.reshape(1, 4 * H)
    wdecT = fcdec_W.T
    bdecr = fcdec_b.reshape(1, 1)
    pdfwT = pdf_W.T
    pdfbr = pdf_b.reshape(1, 1)

    hiddens = jnp.zeros((K * Bn, H), jnp.float32)
    cells = jnp.zeros((K * Bn, H), jnp.float32)
    context = jnp.zeros((Bn, H), jnp.float32)
    for t in range(T):
        hbar, cbar, a = _ka(hiddens, cells, input_encoded, W1T, b1r, w2T, b2r)
        beta = jax.nn.softmax(a.reshape(Bn, T), axis=1)
        context = jnp.einsum('bt,bth->bh', beta, input_encoded)
        h_pre, c_new, proj, lpun = _kb(
            hiddens, cells, context, y_prev[:, t:t + 1],
            eps_all[t], hbar,
            fcwT, fcbr, varWT, varbr, wiTr, bir, WhT, bhr, wdecT, bdecr, pdfwT, pdfbr)
        comp, idx2 = _kc(proj.reshape(K, Bn), lpun.reshape(K, Bn), gum_all[t])
        offs2 = jnp.arange(Bn)
        hiddens = h_pre[(offs2[None, :] + comp * Bn).reshape(-1)]
        cells = c_new[(offs2[None, :] + idx2 * Bn).reshape(-1)]
    y_dec = jnp.mean(hiddens.reshape(K, Bn, H), axis=0) @ fcdec_W.T + fcdec_b
    y_enc = context @ fcenc_W.T + fcenc_b
    return y_dec + y_enc
